# baseline (speedup 1.0000x reference)
import functools

import jax
import jax.numpy as jnp
from jax import lax
from jax.experimental import pallas as pl
from jax.experimental.pallas import tpu as pltpu
from jax.experimental.pallas import tpu_sc as plsc

_BT = 256
_BF = 512
_CB = 256


def _routing_body(logits_ref, pos0_ref, pos1_ref, g0_ref, g1_ref, te_ref,
                  *, bt, nt):
    l = logits_ref[...]
    T, E = l.shape
    ie = lax.broadcasted_iota(jnp.int32, (T, E), 1)
    m1 = jnp.max(l, axis=1, keepdims=True)
    idx1 = jnp.min(jnp.where(l == m1, ie, E), axis=1, keepdims=True)
    sel1 = ie == idx1
    l2 = jnp.where(sel1, -jnp.inf, l)
    m2 = jnp.max(l2, axis=1, keepdims=True)
    idx2 = jnp.min(jnp.where(l2 == m2, ie, E), axis=1, keepdims=True)
    sel2 = ie == idx2
    ed = jnp.exp(m2 - m1)
    g1 = 1.0 / (1.0 + ed)
    g2 = ed * g1

    oh1 = jnp.where(sel1, 1.0, 0.0)
    oh2 = jnp.where(sel2, 1.0, 0.0)
    oh = oh1 + oh2

    nb = T // _CB
    ir = lax.broadcasted_iota(jnp.int32, (_CB, _CB), 0)
    ic = lax.broadcasted_iota(jnp.int32, (_CB, _CB), 1)
    tri = jnp.where(ir > ic, 1.0, 0.0)
    blocks = []
    carry = jnp.zeros((1, E), jnp.float32)
    for b in range(nb):
        blk = oh[b * _CB:(b + 1) * _CB]
        blocks.append(jnp.dot(tri, blk, preferred_element_type=jnp.float32)
                      + carry)
        carry = carry + jnp.sum(blk, axis=0, keepdims=True)
    S = jnp.concatenate(blocks, axis=0)
    counts = carry

    fbt = jnp.float32(bt)
    cpad = jnp.floor((counts + (fbt - 1.0)) / fbt) * fbt
    iu = lax.broadcasted_iota(jnp.int32, (E, E), 0)
    ju = lax.broadcasted_iota(jnp.int32, (E, E), 1)
    ut = jnp.where(iu < ju, 1.0, 0.0)
    off = jnp.dot(cpad, ut, preferred_element_type=jnp.float32)
    off_end = off + cpad

    offb = jnp.broadcast_to(off, (T, E))
    pos0 = jnp.sum(jnp.where(sel1, offb + S, 0.0), axis=1, keepdims=True)
    pos1 = jnp.sum(jnp.where(sel2, offb + S, 0.0), axis=1, keepdims=True)
    pos0_ref[...] = pos0.astype(jnp.int32)
    pos1_ref[...] = pos1.astype(jnp.int32)

    g0_ref[...] = jnp.broadcast_to(g1, (T, 128))
    g1_ref[...] = jnp.broadcast_to(g2, (T, 128))

    tb = lax.broadcasted_iota(jnp.int32, (nt, E), 0).astype(jnp.float32) * fbt
    oeb = jnp.broadcast_to(off_end, (nt, E))
    te = jnp.sum(jnp.where(oeb <= tb, 1, 0), axis=1, keepdims=True)
    te_ref[...] = jnp.minimum(te, E - 1).astype(jnp.int32)


def _make_routing(T, E, nt):
    return pl.pallas_call(
        functools.partial(_routing_body, bt=_BT, nt=nt),
        out_shape=[
            jax.ShapeDtypeStruct((T, 1), jnp.int32),
            jax.ShapeDtypeStruct((T, 1), jnp.int32),
            jax.ShapeDtypeStruct((T, 128), jnp.float32),
            jax.ShapeDtypeStruct((T, 128), jnp.float32),
            jax.ShapeDtypeStruct((nt, 1), jnp.int32),
        ],
    )


def _dispatch_sc(T, H, NP):
    info = plsc.get_sparse_core_info()
    nw = info.num_cores * info.num_subcores
    rows = T // nw
    mesh = plsc.VectorSubcoreMesh(core_axis_name="c", subcore_axis_name="s")

    @functools.partial(
        pl.kernel, mesh=mesh,
        out_type=[jax.ShapeDtypeStruct((NP, H), jnp.float32),
                  jax.ShapeDtypeStruct((NP, 128), jnp.float32)],
        scratch_types=[
            pltpu.VMEM((rows,), jnp.int32),
            pltpu.VMEM((rows,), jnp.int32),
            pltpu.VMEM((rows, H), jnp.float32),
            pltpu.VMEM((rows, 128), jnp.float32),
            pltpu.SemaphoreType.DMA,
        ],
    )
    def k(x_hbm, pos0_hbm, pos1_hbm, g0_hbm, g1_hbm, xs_hbm, gs_hbm,
          idx0_v, idx1_v, x_v, g_v, sem):
        wid = lax.axis_index("s") * info.num_cores + lax.axis_index("c")
        base = wid * rows
        pltpu.sync_copy(pos0_hbm.at[pl.ds(base, rows)], idx0_v)
        pltpu.sync_copy(pos1_hbm.at[pl.ds(base, rows)], idx1_v)
        pltpu.sync_copy(x_hbm.at[pl.ds(base, rows)], x_v)
        pltpu.async_copy(x_v, xs_hbm.at[idx0_v], sem).wait()
        pltpu.async_copy(x_v, xs_hbm.at[idx1_v], sem).wait()
        pltpu.sync_copy(g0_hbm.at[pl.ds(base, rows)], g_v)
        pltpu.async_copy(g_v, gs_hbm.at[idx0_v], sem).wait()
        pltpu.sync_copy(g1_hbm.at[pl.ds(base, rows)], g_v)
        pltpu.async_copy(g_v, gs_hbm.at[idx1_v], sem).wait()

    return k


def _gemm_body(te_ref, x_ref, w1_ref, b1_ref, w2_ref, b2_ref, gs_ref,
               out_ref, acc_ref, *, nf):
    f = pl.program_id(0)
    i = pl.program_id(1)
    bt = x_ref.shape[0]
    isl = pl.ds(i * bt, bt)

    x = x_ref[...].astype(jnp.bfloat16)
    h = jnp.dot(x, w1_ref[0].astype(jnp.bfloat16),
                preferred_element_type=jnp.float32) + b1_ref[0]
    h = jax.nn.gelu(h).astype(jnp.bfloat16)
    part = jnp.dot(h, w2_ref[0].astype(jnp.bfloat16),
                   preferred_element_type=jnp.float32)

    total = part + jnp.where(f == 0, jnp.zeros_like(part), acc_ref[isl, :])
    acc_ref[isl, :] = total

    @pl.when(f == nf - 1)
    def _():
        c = gs_ref[:, :1]
        out_ref[...] = c * (total + b2_ref[0])


def _make_gemm(NP, H, F, E, nt, nf):
    grid_spec = pltpu.PrefetchScalarGridSpec(
        num_scalar_prefetch=1,
        grid=(nf, nt),
        in_specs=[
            pl.BlockSpec((_BT, H), lambda f, i, te: (i, 0)),
            pl.BlockSpec((1, H, _BF), lambda f, i, te: (te[i], 0, f)),
            pl.BlockSpec((1, 1, _BF), lambda f, i, te: (te[i], 0, f)),
            pl.BlockSpec((1, _BF, H), lambda f, i, te: (te[i], f, 0)),
            pl.BlockSpec((1, 1, H), lambda f, i, te: (te[i], 0, 0)),
            pl.BlockSpec((_BT, 128), lambda f, i, te: (i, 0)),
        ],
        out_specs=pl.BlockSpec(
            (_BT, H), lambda f, i, te: (jnp.where(f == nf - 1, i, 0), 0)),
        scratch_shapes=[pltpu.VMEM((NP, H), jnp.float32)],
    )
    return pl.pallas_call(
        functools.partial(_gemm_body, nf=nf),
        grid_spec=grid_spec,
        out_shape=jax.ShapeDtypeStruct((NP, H), jnp.float32),
        compiler_params=pltpu.CompilerParams(
            dimension_semantics=("arbitrary", "arbitrary"),
        ),
    )


def _combine_sc(T, H, NP):
    info = plsc.get_sparse_core_info()
    nw = info.num_cores * info.num_subcores
    rows = T // nw
    ch = 32
    nch = rows // ch
    nsl = H // 16
    mesh = plsc.VectorSubcoreMesh(core_axis_name="c", subcore_axis_name="s")

    @functools.partial(
        pl.kernel, mesh=mesh,
        out_type=jax.ShapeDtypeStruct((T, H), jnp.float32),
        scratch_types=[
            pltpu.VMEM((ch,), jnp.int32),
            pltpu.VMEM((ch, H), jnp.float32),
            pltpu.VMEM((ch, H), jnp.float32),
            pltpu.SemaphoreType.DMA,
        ],
    )
    def k(y_hbm, pos0_hbm, pos1_hbm, out_hbm, idx_v, y_v, o_v, sem):
        wid = lax.axis_index("s") * info.num_cores + lax.axis_index("c")
        for c in range(nch):
            bc = wid * rows + c * ch
            pltpu.sync_copy(pos0_hbm.at[pl.ds(bc, ch)], idx_v)
            pltpu.async_copy(y_hbm.at[idx_v], o_v, sem).wait()
            pltpu.sync_copy(pos1_hbm.at[pl.ds(bc, ch)], idx_v)
            pltpu.async_copy(y_hbm.at[idx_v], y_v, sem).wait()

            def row(r, _):
                def col(kk, _):
                    for u in range(4):
                        sl = pl.ds((kk * 4 + u) * 16, 16)
                        o_v[r, sl] = o_v[r, sl] + y_v[r, sl]
                    return 0
                return lax.fori_loop(0, nsl // 4, col, 0)

            lax.fori_loop(0, ch, row, 0)
            pltpu.sync_copy(o_v, out_hbm.at[pl.ds(bc, ch)])

    return k


def kernel(hidden_states, router_logits, w1, b1, w2, b2):
    T, H = hidden_states.shape
    E = router_logits.shape[1]
    F = w1.shape[2]
    NP = T * 2 + E * _BT
    nt = NP // _BT
    nf = F // _BF

    pos0, pos1, g0, g1, te = _make_routing(T, E, nt)(router_logits)
    pos0 = pos0.reshape(T)
    pos1 = pos1.reshape(T)
    te = te.reshape(nt)

    xs, gs = _dispatch_sc(T, H, NP)(hidden_states, pos0, pos1, g0, g1)

    b1r = b1.reshape(E, 1, F)
    b2r = b2.reshape(E, 1, H)
    y = _make_gemm(NP, H, F, E, nt, nf)(te, xs, w1, b1r, w2, b2r, gs)

    out = _combine_sc(T, H, NP)(y, pos0, pos1)
    return out


# speedup vs baseline: 1.0919x; 1.0919x over previous
import functools

import jax
import jax.numpy as jnp
from jax import lax
from jax.experimental import pallas as pl
from jax.experimental.pallas import tpu as pltpu
from jax.experimental.pallas import tpu_sc as plsc

_BT = 256
_BF = 512
_CB = 256


def _routing_body(logits_ref, pos0_ref, pos1_ref, g0_ref, g1_ref, te_ref,
                  *, bt, nt):
    l = logits_ref[...]
    T, E = l.shape
    ie = lax.broadcasted_iota(jnp.int32, (T, E), 1)
    m1 = jnp.max(l, axis=1, keepdims=True)
    idx1 = jnp.min(jnp.where(l == m1, ie, E), axis=1, keepdims=True)
    sel1 = ie == idx1
    l2 = jnp.where(sel1, -jnp.inf, l)
    m2 = jnp.max(l2, axis=1, keepdims=True)
    idx2 = jnp.min(jnp.where(l2 == m2, ie, E), axis=1, keepdims=True)
    sel2 = ie == idx2
    ed = jnp.exp(m2 - m1)
    g1 = 1.0 / (1.0 + ed)
    g2 = ed * g1

    oh1 = jnp.where(sel1, 1.0, 0.0)
    oh2 = jnp.where(sel2, 1.0, 0.0)
    oh = oh1 + oh2

    nb = T // _CB
    ir = lax.broadcasted_iota(jnp.int32, (_CB, _CB), 0)
    ic = lax.broadcasted_iota(jnp.int32, (_CB, _CB), 1)
    tri = jnp.where(ir > ic, 1.0, 0.0)
    blocks = []
    carry = jnp.zeros((1, E), jnp.float32)
    for b in range(nb):
        blk = oh[b * _CB:(b + 1) * _CB]
        blocks.append(jnp.dot(tri, blk, preferred_element_type=jnp.float32)
                      + carry)
        carry = carry + jnp.sum(blk, axis=0, keepdims=True)
    S = jnp.concatenate(blocks, axis=0)
    counts = carry

    fbt = jnp.float32(bt)
    cpad = jnp.floor((counts + (fbt - 1.0)) / fbt) * fbt
    iu = lax.broadcasted_iota(jnp.int32, (E, E), 0)
    ju = lax.broadcasted_iota(jnp.int32, (E, E), 1)
    ut = jnp.where(iu < ju, 1.0, 0.0)
    off = jnp.dot(cpad, ut, preferred_element_type=jnp.float32)
    off_end = off + cpad

    offb = jnp.broadcast_to(off, (T, E))
    pos0 = jnp.sum(jnp.where(sel1, offb + S, 0.0), axis=1, keepdims=True)
    pos1 = jnp.sum(jnp.where(sel2, offb + S, 0.0), axis=1, keepdims=True)
    pos0_ref[...] = pos0.astype(jnp.int32)
    pos1_ref[...] = pos1.astype(jnp.int32)

    g0_ref[...] = jnp.broadcast_to(g1, (T, 128))
    g1_ref[...] = jnp.broadcast_to(g2, (T, 128))

    tb = lax.broadcasted_iota(jnp.int32, (nt, E), 0).astype(jnp.float32) * fbt
    oeb = jnp.broadcast_to(off_end, (nt, E))
    te = jnp.sum(jnp.where(oeb <= tb, 1, 0), axis=1, keepdims=True)
    te_ref[...] = jnp.minimum(te, E - 1).astype(jnp.int32)


def _make_routing(T, E, nt):
    return pl.pallas_call(
        functools.partial(_routing_body, bt=_BT, nt=nt),
        out_shape=[
            jax.ShapeDtypeStruct((T, 1), jnp.int32),
            jax.ShapeDtypeStruct((T, 1), jnp.int32),
            jax.ShapeDtypeStruct((T, 128), jnp.float32),
            jax.ShapeDtypeStruct((T, 128), jnp.float32),
            jax.ShapeDtypeStruct((nt, 1), jnp.int32),
        ],
    )


def _dispatch_sc(T, H, NP):
    info = plsc.get_sparse_core_info()
    nw = info.num_cores * info.num_subcores
    rows = T // nw
    mesh = plsc.VectorSubcoreMesh(core_axis_name="c", subcore_axis_name="s")

    @functools.partial(
        pl.kernel, mesh=mesh,
        out_type=[jax.ShapeDtypeStruct((NP, H), jnp.float32),
                  jax.ShapeDtypeStruct((NP, 128), jnp.float32)],
        scratch_types=[
            pltpu.VMEM((rows,), jnp.int32),
            pltpu.VMEM((rows,), jnp.int32),
            pltpu.VMEM((rows, H), jnp.float32),
            pltpu.VMEM((rows, 128), jnp.float32),
            pltpu.SemaphoreType.DMA,
        ],
    )
    def k(x_hbm, pos0_hbm, pos1_hbm, g0_hbm, g1_hbm, xs_hbm, gs_hbm,
          idx0_v, idx1_v, x_v, g_v, sem):
        wid = lax.axis_index("s") * info.num_cores + lax.axis_index("c")
        base = wid * rows
        pltpu.sync_copy(pos0_hbm.at[pl.ds(base, rows)], idx0_v)
        pltpu.sync_copy(pos1_hbm.at[pl.ds(base, rows)], idx1_v)
        pltpu.sync_copy(x_hbm.at[pl.ds(base, rows)], x_v)
        pltpu.async_copy(x_v, xs_hbm.at[idx0_v], sem).wait()
        pltpu.async_copy(x_v, xs_hbm.at[idx1_v], sem).wait()
        pltpu.sync_copy(g0_hbm.at[pl.ds(base, rows)], g_v)
        pltpu.async_copy(g_v, gs_hbm.at[idx0_v], sem).wait()
        pltpu.sync_copy(g1_hbm.at[pl.ds(base, rows)], g_v)
        pltpu.async_copy(g_v, gs_hbm.at[idx1_v], sem).wait()

    return k


def _gemm_body(te_ref, x_ref, w1_ref, b1_ref, w2_ref, b2_ref, gs_ref,
               out_ref, acc_ref, *, nf, bt):
    f = pl.program_id(0)
    i = pl.program_id(1)
    isl = pl.ds(i * bt, bt)

    x = x_ref[isl, :]
    h = jnp.dot(x, w1_ref[0], preferred_element_type=jnp.float32) + b1_ref[0]
    h = jax.nn.gelu(h)
    part = jnp.dot(h, w2_ref[0], preferred_element_type=jnp.float32)

    total = part + jnp.where(f == 0, jnp.zeros_like(part), acc_ref[isl, :])
    acc_ref[isl, :] = total

    @pl.when(f == nf - 1)
    def _():
        c = gs_ref[:, :1]
        out_ref[...] = c * (total + b2_ref[0])


def _make_gemm(NP, H, F, E, nt, nf):
    grid_spec = pltpu.PrefetchScalarGridSpec(
        num_scalar_prefetch=1,
        grid=(nf, nt),
        in_specs=[
            pl.BlockSpec((NP, H), lambda f, i, te: (0, 0)),
            pl.BlockSpec((1, H, _BF), lambda f, i, te: (te[i], 0, f)),
            pl.BlockSpec((1, 1, _BF), lambda f, i, te: (te[i], 0, f)),
            pl.BlockSpec((1, _BF, H), lambda f, i, te: (te[i], f, 0)),
            pl.BlockSpec((1, 1, H), lambda f, i, te: (te[i], 0, 0)),
            pl.BlockSpec((_BT, 128), lambda f, i, te: (i, 0)),
        ],
        out_specs=pl.BlockSpec(
            (_BT, H), lambda f, i, te: (jnp.where(f == nf - 1, i, 0), 0)),
        scratch_shapes=[pltpu.VMEM((NP, H), jnp.float32)],
    )
    return pl.pallas_call(
        functools.partial(_gemm_body, nf=nf, bt=_BT),
        grid_spec=grid_spec,
        out_shape=jax.ShapeDtypeStruct((NP, H), jnp.float32),
        compiler_params=pltpu.CompilerParams(
            dimension_semantics=("arbitrary", "arbitrary"),
        ),
    )


def _combine_sc(T, H, NP):
    info = plsc.get_sparse_core_info()
    nw = info.num_cores * info.num_subcores
    rows = T // nw
    ch = 32
    nch = rows // ch
    nsl = H // 16
    mesh = plsc.VectorSubcoreMesh(core_axis_name="c", subcore_axis_name="s")

    @functools.partial(
        pl.kernel, mesh=mesh,
        out_type=jax.ShapeDtypeStruct((T, H), jnp.float32),
        scratch_types=[
            pltpu.VMEM((ch,), jnp.int32),
            pltpu.VMEM((ch, H), jnp.float32),
            pltpu.VMEM((ch, H), jnp.float32),
            pltpu.SemaphoreType.DMA,
        ],
    )
    def k(y_hbm, pos0_hbm, pos1_hbm, out_hbm, idx_v, y_v, o_v, sem):
        wid = lax.axis_index("s") * info.num_cores + lax.axis_index("c")
        for c in range(nch):
            bc = wid * rows + c * ch
            pltpu.sync_copy(pos0_hbm.at[pl.ds(bc, ch)], idx_v)
            pltpu.async_copy(y_hbm.at[idx_v], o_v, sem).wait()
            pltpu.sync_copy(pos1_hbm.at[pl.ds(bc, ch)], idx_v)
            pltpu.async_copy(y_hbm.at[idx_v], y_v, sem).wait()

            def row(r, _):
                def col(kk, _):
                    for u in range(4):
                        sl = pl.ds((kk * 4 + u) * 16, 16)
                        o_v[r, sl] = o_v[r, sl] + y_v[r, sl]
                    return 0
                return lax.fori_loop(0, nsl // 4, col, 0)

            lax.fori_loop(0, ch, row, 0)
            pltpu.sync_copy(o_v, out_hbm.at[pl.ds(bc, ch)])

    return k


def kernel(hidden_states, router_logits, w1, b1, w2, b2):
    T, H = hidden_states.shape
    E = router_logits.shape[1]
    F = w1.shape[2]
    nt = (T * 2 + E * (_BT - 1)) // _BT
    NP = nt * _BT
    nf = F // _BF

    pos0, pos1, g0, g1, te = _make_routing(T, E, nt)(router_logits)
    pos0 = pos0.reshape(T)
    pos1 = pos1.reshape(T)
    te = te.reshape(nt)

    xs, gs = _dispatch_sc(T, H, NP)(hidden_states, pos0, pos1, g0, g1)

    b1r = b1.reshape(E, 1, F)
    b2r = b2.reshape(E, 1, H)
    y = _make_gemm(NP, H, F, E, nt, nf)(te, xs, w1, b1r, w2, b2r, gs)

    out = _combine_sc(T, H, NP)(y, pos0, pos1)
    return out


# speedup vs baseline: 1.2315x; 1.1278x over previous
import functools

import jax
import jax.numpy as jnp
from jax import lax
from jax.experimental import pallas as pl
from jax.experimental.pallas import tpu as pltpu
from jax.experimental.pallas import tpu_sc as plsc

_BT = 256
_BF = 512
_CB = 256


def _routing_body(logits_ref, pos0_ref, pos1_ref, g0_ref, g1_ref, te_ref,
                  grp_ref, *, bt, nt):
    l = logits_ref[...]
    T, E = l.shape
    ie = lax.broadcasted_iota(jnp.int32, (T, E), 1)
    m1 = jnp.max(l, axis=1, keepdims=True)
    idx1 = jnp.min(jnp.where(l == m1, ie, E), axis=1, keepdims=True)
    sel1 = ie == idx1
    l2 = jnp.where(sel1, -jnp.inf, l)
    m2 = jnp.max(l2, axis=1, keepdims=True)
    idx2 = jnp.min(jnp.where(l2 == m2, ie, E), axis=1, keepdims=True)
    sel2 = ie == idx2
    ed = jnp.exp(m2 - m1)
    g1 = 1.0 / (1.0 + ed)
    g2 = ed * g1

    oh1 = jnp.where(sel1, 1.0, 0.0)
    oh2 = jnp.where(sel2, 1.0, 0.0)
    oh = oh1 + oh2

    nb = T // _CB
    ir = lax.broadcasted_iota(jnp.int32, (_CB, _CB), 0)
    ic = lax.broadcasted_iota(jnp.int32, (_CB, _CB), 1)
    tri = jnp.where(ir > ic, 1.0, 0.0)
    blocks = []
    carry = jnp.zeros((1, E), jnp.float32)
    for b in range(nb):
        blk = oh[b * _CB:(b + 1) * _CB]
        blocks.append(jnp.dot(tri, blk, preferred_element_type=jnp.float32)
                      + carry)
        carry = carry + jnp.sum(blk, axis=0, keepdims=True)
    S = jnp.concatenate(blocks, axis=0)
    counts = carry

    fbt = jnp.float32(bt)
    cpad = jnp.floor((counts + (fbt - 1.0)) / fbt) * fbt
    iu = lax.broadcasted_iota(jnp.int32, (E, E), 0)
    ju = lax.broadcasted_iota(jnp.int32, (E, E), 1)
    ut = jnp.where(iu < ju, 1.0, 0.0)
    off = jnp.dot(cpad, ut, preferred_element_type=jnp.float32)
    off_end = off + cpad

    offb = jnp.broadcast_to(off, (T, E))
    pos0 = jnp.sum(jnp.where(sel1, offb + S, 0.0), axis=1, keepdims=True)
    pos1 = jnp.sum(jnp.where(sel2, offb + S, 0.0), axis=1, keepdims=True)
    pos0_ref[...] = pos0.astype(jnp.int32)
    pos1_ref[...] = pos1.astype(jnp.int32)

    g0_ref[...] = jnp.broadcast_to(g1, (T, 128))
    g1_ref[...] = jnp.broadcast_to(g2, (T, 128))

    tb = lax.broadcasted_iota(jnp.int32, (nt, E), 0).astype(jnp.float32) * fbt
    oeb = jnp.broadcast_to(off_end, (nt, E))
    te = jnp.sum(jnp.where(oeb <= tb, 1, 0), axis=1, keepdims=True)
    te = jnp.minimum(te, E - 1)
    te_ref[...] = te.astype(jnp.int32)

    tprev = jnp.concatenate([te[:1], te[:-1]], axis=0)
    chg = jnp.where(te != tprev, 1.0, 0.0)
    gr = lax.broadcasted_iota(jnp.int32, (nt, nt), 0)
    gc = lax.broadcasted_iota(jnp.int32, (nt, nt), 1)
    tri_i = jnp.where(gr >= gc, 1.0, 0.0)
    grp = jnp.dot(tri_i, chg, preferred_element_type=jnp.float32)
    grp_ref[...] = grp.astype(jnp.int32)


def _make_routing(T, E, nt):
    return pl.pallas_call(
        functools.partial(_routing_body, bt=_BT, nt=nt),
        out_shape=[
            jax.ShapeDtypeStruct((T, 1), jnp.int32),
            jax.ShapeDtypeStruct((T, 1), jnp.int32),
            jax.ShapeDtypeStruct((T, 128), jnp.float32),
            jax.ShapeDtypeStruct((T, 128), jnp.float32),
            jax.ShapeDtypeStruct((nt, 1), jnp.int32),
            jax.ShapeDtypeStruct((nt, 1), jnp.int32),
        ],
    )


def _dispatch_sc(T, H, NP):
    info = plsc.get_sparse_core_info()
    nw = info.num_cores * info.num_subcores
    rows = T // nw
    mesh = plsc.VectorSubcoreMesh(core_axis_name="c", subcore_axis_name="s")

    @functools.partial(
        pl.kernel, mesh=mesh,
        out_type=[jax.ShapeDtypeStruct((NP, H), jnp.float32),
                  jax.ShapeDtypeStruct((NP, 128), jnp.float32)],
        scratch_types=[
            pltpu.VMEM((rows,), jnp.int32),
            pltpu.VMEM((rows,), jnp.int32),
            pltpu.VMEM((rows, H), jnp.float32),
            pltpu.VMEM((rows, 128), jnp.float32),
            pltpu.SemaphoreType.DMA,
        ],
    )
    def k(x_hbm, pos0_hbm, pos1_hbm, g0_hbm, g1_hbm, xs_hbm, gs_hbm,
          idx0_v, idx1_v, x_v, g_v, sem):
        wid = lax.axis_index("s") * info.num_cores + lax.axis_index("c")
        base = wid * rows
        pltpu.sync_copy(pos0_hbm.at[pl.ds(base, rows)], idx0_v)
        pltpu.sync_copy(pos1_hbm.at[pl.ds(base, rows)], idx1_v)
        pltpu.sync_copy(x_hbm.at[pl.ds(base, rows)], x_v)
        pltpu.async_copy(x_v, xs_hbm.at[idx0_v], sem).wait()
        pltpu.async_copy(x_v, xs_hbm.at[idx1_v], sem).wait()
        pltpu.sync_copy(g0_hbm.at[pl.ds(base, rows)], g_v)
        pltpu.async_copy(g_v, gs_hbm.at[idx0_v], sem).wait()
        pltpu.sync_copy(g1_hbm.at[pl.ds(base, rows)], g_v)
        pltpu.async_copy(g_v, gs_hbm.at[idx1_v], sem).wait()

    return k


_NBUF = 4
_LOOK = 2


def _gemm_body(te_ref, grp_ref, x_ref, b1_ref, b2_ref, gs_ref,
               w1_ref, w2_ref, out_ref, acc_ref, wb1_ref, wb2_ref, sems,
               *, nf, nt, bt, bf):
    f = pl.program_id(0)
    i = pl.program_id(1)
    isl = pl.ds(i * bt, bt)
    G = grp_ref[nt - 1] + 1
    s = f * nt + i
    S = nf * nt

    def dma_pair(f2, i2, slot):
        e2 = te_ref[i2]
        c1 = pltpu.make_async_copy(
            w1_ref.at[e2, :, pl.ds(f2 * bf, bf)], wb1_ref.at[slot],
            sems.at[slot])
        c2 = pltpu.make_async_copy(
            w2_ref.at[e2, pl.ds(f2 * bf, bf), :], wb2_ref.at[slot],
            sems.at[slot])
        return c1, c2

    def issue(f2, i2, dk2):
        slot = lax.rem(dk2, _NBUF)
        c1, c2 = dma_pair(f2, i2, slot)
        c1.start()
        c2.start()

    @pl.when(s == 0)
    def _():
        issue(0, 0, grp_ref[0])

        @pl.when(grp_ref[1] != grp_ref[0])
        def _():
            issue(0, 1, grp_ref[1])

    s2 = s + _LOOK

    @pl.when(s2 < S)
    def _():
        f2 = s2 // nt
        i2 = lax.rem(s2, nt)
        dk2 = f2 * G + grp_ref[i2]
        s3 = s2 - 1
        f3 = s3 // nt
        i3 = lax.rem(s3, nt)
        dk3 = f3 * G + grp_ref[i3]

        @pl.when(dk2 != dk3)
        def _():
            issue(f2, i2, dk2)

    dk = f * G + grp_ref[i]
    slot = lax.rem(dk, _NBUF)
    sp = jnp.maximum(s - 1, 0)
    fp = sp // nt
    ip = lax.rem(sp, nt)
    dkp = jnp.where(s == 0, -1, fp * G + grp_ref[ip])

    @pl.when(dk != dkp)
    def _():
        c1, c2 = dma_pair(f, i, slot)
        c1.wait()
        c2.wait()

    x = x_ref[...]
    h = jnp.dot(x, wb1_ref[slot], preferred_element_type=jnp.float32) \
        + b1_ref[0]
    h = jax.nn.gelu(h)
    part = jnp.dot(h, wb2_ref[slot], preferred_element_type=jnp.float32)

    total = part + jnp.where(f == 0, jnp.zeros_like(part), acc_ref[isl, :])
    acc_ref[isl, :] = total

    @pl.when(f == nf - 1)
    def _():
        c = gs_ref[:, :1]
        out_ref[...] = c * (total + b2_ref[0])


def _make_gemm(NP, H, F, E, nt, nf):
    grid_spec = pltpu.PrefetchScalarGridSpec(
        num_scalar_prefetch=2,
        grid=(nf, nt),
        in_specs=[
            pl.BlockSpec((_BT, H), lambda f, i, te, g: (i, 0)),
            pl.BlockSpec((1, 1, _BF), lambda f, i, te, g: (te[i], 0, f)),
            pl.BlockSpec((1, 1, H), lambda f, i, te, g: (te[i], 0, 0)),
            pl.BlockSpec((_BT, 128), lambda f, i, te, g: (i, 0)),
            pl.BlockSpec(memory_space=pl.ANY),
            pl.BlockSpec(memory_space=pl.ANY),
        ],
        out_specs=pl.BlockSpec(
            (_BT, H), lambda f, i, te, g: (jnp.where(f == nf - 1, i, 0), 0)),
        scratch_shapes=[
            pltpu.VMEM((NP, H), jnp.float32),
            pltpu.VMEM((_NBUF, H, _BF), jnp.float32),
            pltpu.VMEM((_NBUF, _BF, H), jnp.float32),
            pltpu.SemaphoreType.DMA((_NBUF,)),
        ],
    )
    return pl.pallas_call(
        functools.partial(_gemm_body, nf=nf, nt=nt, bt=_BT, bf=_BF),
        grid_spec=grid_spec,
        out_shape=jax.ShapeDtypeStruct((NP, H), jnp.float32),
        compiler_params=pltpu.CompilerParams(
            dimension_semantics=("arbitrary", "arbitrary"),
        ),
    )


def _combine_sc(T, H, NP):
    info = plsc.get_sparse_core_info()
    nw = info.num_cores * info.num_subcores
    rows = T // nw
    ch = 32
    nch = rows // ch
    nsl = H // 16
    mesh = plsc.VectorSubcoreMesh(core_axis_name="c", subcore_axis_name="s")

    @functools.partial(
        pl.kernel, mesh=mesh,
        out_type=jax.ShapeDtypeStruct((T, H), jnp.float32),
        scratch_types=[
            pltpu.VMEM((ch,), jnp.int32),
            pltpu.VMEM((ch, H), jnp.float32),
            pltpu.VMEM((ch, H), jnp.float32),
            pltpu.SemaphoreType.DMA,
        ],
    )
    def k(y_hbm, pos0_hbm, pos1_hbm, out_hbm, idx_v, y_v, o_v, sem):
        wid = lax.axis_index("s") * info.num_cores + lax.axis_index("c")
        for c in range(nch):
            bc = wid * rows + c * ch
            pltpu.sync_copy(pos0_hbm.at[pl.ds(bc, ch)], idx_v)
            pltpu.async_copy(y_hbm.at[idx_v], o_v, sem).wait()
            pltpu.sync_copy(pos1_hbm.at[pl.ds(bc, ch)], idx_v)
            pltpu.async_copy(y_hbm.at[idx_v], y_v, sem).wait()

            def row(r, _):
                def col(kk, _):
                    for u in range(4):
                        sl = pl.ds((kk * 4 + u) * 16, 16)
                        o_v[r, sl] = o_v[r, sl] + y_v[r, sl]
                    return 0
                return lax.fori_loop(0, nsl // 4, col, 0)

            lax.fori_loop(0, ch, row, 0)
            pltpu.sync_copy(o_v, out_hbm.at[pl.ds(bc, ch)])

    return k


def kernel(hidden_states, router_logits, w1, b1, w2, b2):
    T, H = hidden_states.shape
    E = router_logits.shape[1]
    F = w1.shape[2]
    nt = (T * 2 + E * (_BT - 1)) // _BT
    NP = nt * _BT
    nf = F // _BF

    pos0, pos1, g0, g1, te, grp = _make_routing(T, E, nt)(router_logits)
    pos0 = pos0.reshape(T)
    pos1 = pos1.reshape(T)
    te = te.reshape(nt)
    grp = grp.reshape(nt)

    xs, gs = _dispatch_sc(T, H, NP)(hidden_states, pos0, pos1, g0, g1)

    b1r = b1.reshape(E, 1, F)
    b2r = b2.reshape(E, 1, H)
    y = _make_gemm(NP, H, F, E, nt, nf)(te, grp, xs, b1r, b2r, gs, w1, w2)

    out = _combine_sc(T, H, NP)(y, pos0, pos1)
    return out


# speedup vs baseline: 1.2343x; 1.0023x over previous
import functools

import jax
import jax.numpy as jnp
from jax import lax
from jax.experimental import pallas as pl
from jax.experimental.pallas import tpu as pltpu
from jax.experimental.pallas import tpu_sc as plsc

_BT = 256
_BF = 512
_CB = 256


def _routing_body(logits_ref, pos0_ref, pos1_ref, g0_ref, g1_ref, te_ref,
                  grp_ref, *, bt, nt):
    l = logits_ref[...]
    T, E = l.shape
    ie = lax.broadcasted_iota(jnp.int32, (T, E), 1)
    m1 = jnp.max(l, axis=1, keepdims=True)
    idx1 = jnp.min(jnp.where(l == m1, ie, E), axis=1, keepdims=True)
    sel1 = ie == idx1
    l2 = jnp.where(sel1, -jnp.inf, l)
    m2 = jnp.max(l2, axis=1, keepdims=True)
    idx2 = jnp.min(jnp.where(l2 == m2, ie, E), axis=1, keepdims=True)
    sel2 = ie == idx2
    ed = jnp.exp(m2 - m1)
    g1 = 1.0 / (1.0 + ed)
    g2 = ed * g1

    oh1 = jnp.where(sel1, 1.0, 0.0)
    oh2 = jnp.where(sel2, 1.0, 0.0)
    oh = oh1 + oh2

    nb = T // _CB
    ir = lax.broadcasted_iota(jnp.int32, (_CB, _CB), 0)
    ic = lax.broadcasted_iota(jnp.int32, (_CB, _CB), 1)
    tri = jnp.where(ir > ic, 1.0, 0.0)
    blocks = []
    carry = jnp.zeros((1, E), jnp.float32)
    for b in range(nb):
        blk = oh[b * _CB:(b + 1) * _CB]
        blocks.append(jnp.dot(tri, blk, preferred_element_type=jnp.float32)
                      + carry)
        carry = carry + jnp.sum(blk, axis=0, keepdims=True)
    S = jnp.concatenate(blocks, axis=0)
    counts = carry

    fbt = jnp.float32(bt)
    cpad = jnp.floor((counts + (fbt - 1.0)) / fbt) * fbt
    iu = lax.broadcasted_iota(jnp.int32, (E, E), 0)
    ju = lax.broadcasted_iota(jnp.int32, (E, E), 1)
    ut = jnp.where(iu < ju, 1.0, 0.0)
    off = jnp.dot(cpad, ut, preferred_element_type=jnp.float32)
    off_end = off + cpad

    offb = jnp.broadcast_to(off, (T, E))
    pos0 = jnp.sum(jnp.where(sel1, offb + S, 0.0), axis=1, keepdims=True)
    pos1 = jnp.sum(jnp.where(sel2, offb + S, 0.0), axis=1, keepdims=True)
    pos0_ref[...] = pos0.astype(jnp.int32)
    pos1_ref[...] = pos1.astype(jnp.int32)

    g0_ref[...] = jnp.broadcast_to(g1, (T, 128))
    g1_ref[...] = jnp.broadcast_to(g2, (T, 128))

    tb = lax.broadcasted_iota(jnp.int32, (nt, E), 0).astype(jnp.float32) * fbt
    oeb = jnp.broadcast_to(off_end, (nt, E))
    te = jnp.sum(jnp.where(oeb <= tb, 1, 0), axis=1, keepdims=True)
    te = jnp.minimum(te, E - 1)
    te_ref[...] = te.astype(jnp.int32)

    tprev = jnp.concatenate([te[:1], te[:-1]], axis=0)
    chg = jnp.where(te != tprev, 1.0, 0.0)
    gr = lax.broadcasted_iota(jnp.int32, (nt, nt), 0)
    gc = lax.broadcasted_iota(jnp.int32, (nt, nt), 1)
    tri_i = jnp.where(gr >= gc, 1.0, 0.0)
    grp = jnp.dot(tri_i, chg, preferred_element_type=jnp.float32)
    grp_ref[...] = grp.astype(jnp.int32)


def _make_routing(T, E, nt):
    return pl.pallas_call(
        functools.partial(_routing_body, bt=_BT, nt=nt),
        out_shape=[
            jax.ShapeDtypeStruct((T, 1), jnp.int32),
            jax.ShapeDtypeStruct((T, 1), jnp.int32),
            jax.ShapeDtypeStruct((T, 128), jnp.float32),
            jax.ShapeDtypeStruct((T, 128), jnp.float32),
            jax.ShapeDtypeStruct((nt, 1), jnp.int32),
            jax.ShapeDtypeStruct((nt, 1), jnp.int32),
        ],
    )


def _dispatch_sc(T, H, NP):
    info = plsc.get_sparse_core_info()
    nw = info.num_cores * info.num_subcores
    rows = T // nw
    mesh = plsc.VectorSubcoreMesh(core_axis_name="c", subcore_axis_name="s")

    @functools.partial(
        pl.kernel, mesh=mesh,
        out_type=[jax.ShapeDtypeStruct((NP, H), jnp.float32),
                  jax.ShapeDtypeStruct((NP, 128), jnp.float32)],
        scratch_types=[
            pltpu.VMEM((rows,), jnp.int32),
            pltpu.VMEM((rows,), jnp.int32),
            pltpu.VMEM((rows, H), jnp.float32),
            pltpu.VMEM((rows, 128), jnp.float32),
            pltpu.SemaphoreType.DMA,
        ],
    )
    def k(x_hbm, pos0_hbm, pos1_hbm, g0_hbm, g1_hbm, xs_hbm, gs_hbm,
          idx0_v, idx1_v, x_v, g_v, sem):
        wid = lax.axis_index("s") * info.num_cores + lax.axis_index("c")
        base = wid * rows
        pltpu.sync_copy(pos0_hbm.at[pl.ds(base, rows)], idx0_v)
        pltpu.sync_copy(pos1_hbm.at[pl.ds(base, rows)], idx1_v)
        pltpu.sync_copy(x_hbm.at[pl.ds(base, rows)], x_v)
        pltpu.async_copy(x_v, xs_hbm.at[idx0_v], sem).wait()
        pltpu.async_copy(x_v, xs_hbm.at[idx1_v], sem).wait()
        pltpu.sync_copy(g0_hbm.at[pl.ds(base, rows)], g_v)
        pltpu.async_copy(g_v, gs_hbm.at[idx0_v], sem).wait()
        pltpu.sync_copy(g1_hbm.at[pl.ds(base, rows)], g_v)
        pltpu.async_copy(g_v, gs_hbm.at[idx1_v], sem).wait()

    return k


_NBUF = 4
_LOOK = 3


def _gemm_body(te_ref, grp_ref, x_ref, b1_ref, b2_ref, gs_ref,
               w1_ref, w2_ref, out_ref, acc_ref, wb1_ref, wb2_ref, sems,
               *, nf, nt, bt, bf):
    f = pl.program_id(0)
    i = pl.program_id(1)
    isl = pl.ds(i * bt, bt)
    G = grp_ref[nt - 1] + 1
    s = f * nt + i
    S = nf * nt

    def dma_pair(f2, i2, slot):
        e2 = te_ref[i2]
        c1 = pltpu.make_async_copy(
            w1_ref.at[e2, :, pl.ds(f2 * bf, bf)], wb1_ref.at[slot],
            sems.at[slot])
        c2 = pltpu.make_async_copy(
            w2_ref.at[e2, pl.ds(f2 * bf, bf), :], wb2_ref.at[slot],
            sems.at[slot])
        return c1, c2

    def issue(f2, i2, dk2):
        slot = lax.rem(dk2, _NBUF)
        c1, c2 = dma_pair(f2, i2, slot)
        c1.start()
        c2.start()

    @pl.when(s == 0)
    def _():
        issue(0, 0, grp_ref[0])
        for u in range(1, _LOOK):
            @pl.when(grp_ref[u] != grp_ref[u - 1])
            def _(u=u):
                issue(0, u, grp_ref[u])

    s2 = s + _LOOK

    @pl.when(s2 < S)
    def _():
        f2 = s2 // nt
        i2 = lax.rem(s2, nt)
        dk2 = f2 * G + grp_ref[i2]
        s3 = s2 - 1
        f3 = s3 // nt
        i3 = lax.rem(s3, nt)
        dk3 = f3 * G + grp_ref[i3]

        @pl.when(dk2 != dk3)
        def _():
            issue(f2, i2, dk2)

    dk = f * G + grp_ref[i]
    slot = lax.rem(dk, _NBUF)
    sp = jnp.maximum(s - 1, 0)
    fp = sp // nt
    ip = lax.rem(sp, nt)
    dkp = jnp.where(s == 0, -1, fp * G + grp_ref[ip])

    @pl.when(dk != dkp)
    def _():
        c1, c2 = dma_pair(f, i, slot)
        c1.wait()
        c2.wait()

    x = x_ref[...]
    h = jnp.dot(x, wb1_ref[slot], preferred_element_type=jnp.float32) \
        + b1_ref[0]
    h = jax.nn.gelu(h)
    part = jnp.dot(h, wb2_ref[slot], preferred_element_type=jnp.float32)

    total = part + jnp.where(f == 0, jnp.zeros_like(part), acc_ref[isl, :])
    acc_ref[isl, :] = total

    @pl.when(f == nf - 1)
    def _():
        c = gs_ref[:, :1]
        out_ref[...] = c * (total + b2_ref[0])


def _make_gemm(NP, H, F, E, nt, nf):
    grid_spec = pltpu.PrefetchScalarGridSpec(
        num_scalar_prefetch=2,
        grid=(nf, nt),
        in_specs=[
            pl.BlockSpec((_BT, H), lambda f, i, te, g: (i, 0)),
            pl.BlockSpec((1, 1, _BF), lambda f, i, te, g: (te[i], 0, f)),
            pl.BlockSpec((1, 1, H), lambda f, i, te, g: (te[i], 0, 0)),
            pl.BlockSpec((_BT, 128), lambda f, i, te, g: (i, 0)),
            pl.BlockSpec(memory_space=pl.ANY),
            pl.BlockSpec(memory_space=pl.ANY),
        ],
        out_specs=pl.BlockSpec(
            (_BT, H), lambda f, i, te, g: (jnp.where(f == nf - 1, i, 0), 0)),
        scratch_shapes=[
            pltpu.VMEM((NP, H), jnp.float32),
            pltpu.VMEM((_NBUF, H, _BF), jnp.float32),
            pltpu.VMEM((_NBUF, _BF, H), jnp.float32),
            pltpu.SemaphoreType.DMA((_NBUF,)),
        ],
    )
    return pl.pallas_call(
        functools.partial(_gemm_body, nf=nf, nt=nt, bt=_BT, bf=_BF),
        grid_spec=grid_spec,
        out_shape=jax.ShapeDtypeStruct((NP, H), jnp.float32),
        compiler_params=pltpu.CompilerParams(
            dimension_semantics=("arbitrary", "arbitrary"),
        ),
    )


def _combine_sc(T, H, NP):
    info = plsc.get_sparse_core_info()
    nw = info.num_cores * info.num_subcores
    rows = T // nw
    ch = 32
    nch = rows // ch
    nsl = H // 16
    mesh = plsc.VectorSubcoreMesh(core_axis_name="c", subcore_axis_name="s")

    @functools.partial(
        pl.kernel, mesh=mesh,
        out_type=jax.ShapeDtypeStruct((T, H), jnp.float32),
        scratch_types=[
            pltpu.VMEM((ch,), jnp.int32),
            pltpu.VMEM((ch, H), jnp.float32),
            pltpu.VMEM((ch, H), jnp.float32),
            pltpu.SemaphoreType.DMA,
        ],
    )
    def k(y_hbm, pos0_hbm, pos1_hbm, out_hbm, idx_v, y_v, o_v, sem):
        wid = lax.axis_index("s") * info.num_cores + lax.axis_index("c")
        for c in range(nch):
            bc = wid * rows + c * ch
            pltpu.sync_copy(pos0_hbm.at[pl.ds(bc, ch)], idx_v)
            pltpu.async_copy(y_hbm.at[idx_v], o_v, sem).wait()
            pltpu.sync_copy(pos1_hbm.at[pl.ds(bc, ch)], idx_v)
            pltpu.async_copy(y_hbm.at[idx_v], y_v, sem).wait()

            def row(r, _):
                def col(kk, _):
                    for u in range(4):
                        sl = pl.ds((kk * 4 + u) * 16, 16)
                        o_v[r, sl] = o_v[r, sl] + y_v[r, sl]
                    return 0
                return lax.fori_loop(0, nsl // 4, col, 0)

            lax.fori_loop(0, ch, row, 0)
            pltpu.sync_copy(o_v, out_hbm.at[pl.ds(bc, ch)])

    return k


def kernel(hidden_states, router_logits, w1, b1, w2, b2):
    T, H = hidden_states.shape
    E = router_logits.shape[1]
    F = w1.shape[2]
    nt = (T * 2 + E * (_BT - 1)) // _BT
    NP = nt * _BT
    nf = F // _BF

    pos0, pos1, g0, g1, te, grp = _make_routing(T, E, nt)(router_logits)
    pos0 = pos0.reshape(T)
    pos1 = pos1.reshape(T)
    te = te.reshape(nt)
    grp = grp.reshape(nt)

    xs, gs = _dispatch_sc(T, H, NP)(hidden_states, pos0, pos1, g0, g1)

    b1r = b1.reshape(E, 1, F)
    b2r = b2.reshape(E, 1, H)
    y = _make_gemm(NP, H, F, E, nt, nf)(te, grp, xs, b1r, b2r, gs, w1, w2)

    out = _combine_sc(T, H, NP)(y, pos0, pos1)
    return out


# speedup vs baseline: 1.3433x; 1.0883x over previous
import functools

import jax
import jax.numpy as jnp
from jax import lax
from jax.experimental import pallas as pl
from jax.experimental.pallas import tpu as pltpu
from jax.experimental.pallas import tpu_sc as plsc

_BT = 256
_BF = 512
_CB = 256


def _routing_body(logits_ref, pos0_ref, pos1_ref, g0_ref, g1_ref, te_ref,
                  grp_ref, *, bt, nt):
    l = logits_ref[...]
    T, E = l.shape
    ie = lax.broadcasted_iota(jnp.int32, (T, E), 1)
    m1 = jnp.max(l, axis=1, keepdims=True)
    idx1 = jnp.min(jnp.where(l == m1, ie, E), axis=1, keepdims=True)
    sel1 = ie == idx1
    l2 = jnp.where(sel1, -jnp.inf, l)
    m2 = jnp.max(l2, axis=1, keepdims=True)
    idx2 = jnp.min(jnp.where(l2 == m2, ie, E), axis=1, keepdims=True)
    sel2 = ie == idx2
    ed = jnp.exp(m2 - m1)
    g1 = 1.0 / (1.0 + ed)
    g2 = ed * g1

    oh1 = jnp.where(sel1, 1.0, 0.0)
    oh2 = jnp.where(sel2, 1.0, 0.0)
    oh = oh1 + oh2

    nb = T // _CB
    ir = lax.broadcasted_iota(jnp.int32, (_CB, _CB), 0)
    ic = lax.broadcasted_iota(jnp.int32, (_CB, _CB), 1)
    tri = jnp.where(ir > ic, 1.0, 0.0)
    blocks = []
    carry = jnp.zeros((1, E), jnp.float32)
    for b in range(nb):
        blk = oh[b * _CB:(b + 1) * _CB]
        blocks.append(jnp.dot(tri, blk, preferred_element_type=jnp.float32)
                      + carry)
        carry = carry + jnp.sum(blk, axis=0, keepdims=True)
    S = jnp.concatenate(blocks, axis=0)
    counts = carry

    fbt = jnp.float32(bt)
    cpad = jnp.floor((counts + (fbt - 1.0)) / fbt) * fbt
    iu = lax.broadcasted_iota(jnp.int32, (E, E), 0)
    ju = lax.broadcasted_iota(jnp.int32, (E, E), 1)
    ut = jnp.where(iu < ju, 1.0, 0.0)
    off = jnp.dot(cpad, ut, preferred_element_type=jnp.float32)
    off_end = off + cpad

    offb = jnp.broadcast_to(off, (T, E))
    pos0 = jnp.sum(jnp.where(sel1, offb + S, 0.0), axis=1, keepdims=True)
    pos1 = jnp.sum(jnp.where(sel2, offb + S, 0.0), axis=1, keepdims=True)
    pos0_ref[...] = pos0.astype(jnp.int32)
    pos1_ref[...] = pos1.astype(jnp.int32)

    g0_ref[...] = jnp.broadcast_to(g1, (T, 128))
    g1_ref[...] = jnp.broadcast_to(g2, (T, 128))

    tb = lax.broadcasted_iota(jnp.int32, (nt, E), 0).astype(jnp.float32) * fbt
    oeb = jnp.broadcast_to(off_end, (nt, E))
    te = jnp.sum(jnp.where(oeb <= tb, 1, 0), axis=1, keepdims=True)
    te = jnp.minimum(te, E - 1)
    te_ref[...] = te.astype(jnp.int32)

    tprev = jnp.concatenate([te[:1], te[:-1]], axis=0)
    chg = jnp.where(te != tprev, 1.0, 0.0)
    gr = lax.broadcasted_iota(jnp.int32, (nt, nt), 0)
    gc = lax.broadcasted_iota(jnp.int32, (nt, nt), 1)
    tri_i = jnp.where(gr >= gc, 1.0, 0.0)
    grp = jnp.dot(tri_i, chg, preferred_element_type=jnp.float32)
    grp_ref[...] = grp.astype(jnp.int32)


def _make_routing(T, E, nt):
    return pl.pallas_call(
        functools.partial(_routing_body, bt=_BT, nt=nt),
        out_shape=[
            jax.ShapeDtypeStruct((T, 1), jnp.int32),
            jax.ShapeDtypeStruct((T, 1), jnp.int32),
            jax.ShapeDtypeStruct((T, 128), jnp.float32),
            jax.ShapeDtypeStruct((T, 128), jnp.float32),
            jax.ShapeDtypeStruct((nt, 1), jnp.int32),
            jax.ShapeDtypeStruct((nt, 1), jnp.int32),
        ],
    )


def _dispatch_sc(T, H, NP):
    info = plsc.get_sparse_core_info()
    nw = info.num_cores * info.num_subcores
    rows = T // nw
    mesh = plsc.VectorSubcoreMesh(core_axis_name="c", subcore_axis_name="s")

    @functools.partial(
        pl.kernel, mesh=mesh,
        out_type=[jax.ShapeDtypeStruct((NP, H), jnp.float32),
                  jax.ShapeDtypeStruct((NP, 128), jnp.float32)],
        scratch_types=[
            pltpu.VMEM((rows,), jnp.int32),
            pltpu.VMEM((rows,), jnp.int32),
            pltpu.VMEM((rows, H), jnp.float32),
            pltpu.VMEM((rows, 128), jnp.float32),
            pltpu.SemaphoreType.DMA,
        ],
    )
    def k(x_hbm, pos0_hbm, pos1_hbm, g0_hbm, g1_hbm, xs_hbm, gs_hbm,
          idx0_v, idx1_v, x_v, g_v, sem):
        wid = lax.axis_index("s") * info.num_cores + lax.axis_index("c")
        base = wid * rows
        pltpu.sync_copy(pos0_hbm.at[pl.ds(base, rows)], idx0_v)
        pltpu.sync_copy(pos1_hbm.at[pl.ds(base, rows)], idx1_v)
        pltpu.sync_copy(x_hbm.at[pl.ds(base, rows)], x_v)
        pltpu.async_copy(x_v, xs_hbm.at[idx0_v], sem).wait()
        pltpu.async_copy(x_v, xs_hbm.at[idx1_v], sem).wait()
        pltpu.sync_copy(g0_hbm.at[pl.ds(base, rows)], g_v)
        pltpu.async_copy(g_v, gs_hbm.at[idx0_v], sem).wait()
        pltpu.sync_copy(g1_hbm.at[pl.ds(base, rows)], g_v)
        pltpu.async_copy(g_v, gs_hbm.at[idx1_v], sem).wait()

    return k


_NBUF = 4
_LOOK = 3


def _gemm_body(te_ref, grp_ref, x_ref, b1_ref, b2_ref, gs_ref,
               w1_ref, w2_ref, out_ref, acc_ref, xb_ref, wb1_ref, wb2_ref,
               sems, *, nf, nt, bt, bf):
    f = pl.program_id(0)
    i = pl.program_id(1)
    isl = pl.ds(i * bt, bt)
    G = grp_ref[nt - 1] + 1
    s = f * nt + i
    S = nf * nt

    def dma_pair(f2, i2, slot):
        e2 = te_ref[i2]
        c1 = pltpu.make_async_copy(
            w1_ref.at[e2, :, pl.ds(f2 * bf, bf)], wb1_ref.at[slot],
            sems.at[slot])
        c2 = pltpu.make_async_copy(
            w2_ref.at[e2, pl.ds(f2 * bf, bf), :], wb2_ref.at[slot],
            sems.at[slot])
        return c1, c2

    def issue(f2, i2, dk2):
        slot = lax.rem(dk2, _NBUF)
        c1, c2 = dma_pair(f2, i2, slot)
        c1.start()
        c2.start()

    @pl.when(s == 0)
    def _():
        issue(0, 0, grp_ref[0])
        for u in range(1, _LOOK):
            @pl.when(grp_ref[u] != grp_ref[u - 1])
            def _(u=u):
                issue(0, u, grp_ref[u])

    s2 = s + _LOOK

    @pl.when(s2 < S)
    def _():
        f2 = s2 // nt
        i2 = lax.rem(s2, nt)
        dk2 = f2 * G + grp_ref[i2]
        s3 = s2 - 1
        f3 = s3 // nt
        i3 = lax.rem(s3, nt)
        dk3 = f3 * G + grp_ref[i3]

        @pl.when(dk2 != dk3)
        def _():
            issue(f2, i2, dk2)

    dk = f * G + grp_ref[i]
    slot = lax.rem(dk, _NBUF)
    sp = jnp.maximum(s - 1, 0)
    fp = sp // nt
    ip = lax.rem(sp, nt)
    dkp = jnp.where(s == 0, -1, fp * G + grp_ref[ip])

    @pl.when(dk != dkp)
    def _():
        c1, c2 = dma_pair(f, i, slot)
        c1.wait()
        c2.wait()

    @pl.when(f == 0)
    def _():
        xb_ref[isl, :] = x_ref[...].astype(jnp.bfloat16)

    x = xb_ref[isl, :]
    h = jnp.dot(x, wb1_ref[slot], preferred_element_type=jnp.float32) \
        + b1_ref[0]
    h = jax.nn.gelu(h)
    part = jnp.dot(h, wb2_ref[slot], preferred_element_type=jnp.float32)

    total = part + jnp.where(f == 0, jnp.zeros_like(part), acc_ref[isl, :])
    acc_ref[isl, :] = total

    @pl.when(f == nf - 1)
    def _():
        c = gs_ref[:, :1]
        out_ref[...] = c * (total + b2_ref[0])


def _make_gemm(NP, H, F, E, nt, nf):
    grid_spec = pltpu.PrefetchScalarGridSpec(
        num_scalar_prefetch=2,
        grid=(nf, nt),
        in_specs=[
            pl.BlockSpec(
                (_BT, H),
                lambda f, i, te, g: (jnp.where(f == 0, i, 0), 0)),
            pl.BlockSpec((1, 1, _BF), lambda f, i, te, g: (te[i], 0, f)),
            pl.BlockSpec((1, 1, H), lambda f, i, te, g: (te[i], 0, 0)),
            pl.BlockSpec(
                (_BT, 128),
                lambda f, i, te, g: (jnp.where(f == nf - 1, i, 0), 0)),
            pl.BlockSpec(memory_space=pl.ANY),
            pl.BlockSpec(memory_space=pl.ANY),
        ],
        out_specs=pl.BlockSpec(
            (_BT, H), lambda f, i, te, g: (jnp.where(f == nf - 1, i, 0), 0)),
        scratch_shapes=[
            pltpu.VMEM((NP, H), jnp.float32),
            pltpu.VMEM((NP, H), jnp.bfloat16),
            pltpu.VMEM((_NBUF, H, _BF), jnp.float32),
            pltpu.VMEM((_NBUF, _BF, H), jnp.float32),
            pltpu.SemaphoreType.DMA((_NBUF,)),
        ],
    )
    return pl.pallas_call(
        functools.partial(_gemm_body, nf=nf, nt=nt, bt=_BT, bf=_BF),
        grid_spec=grid_spec,
        out_shape=jax.ShapeDtypeStruct((NP, H), jnp.float32),
        compiler_params=pltpu.CompilerParams(
            dimension_semantics=("arbitrary", "arbitrary"),
        ),
    )


def _combine_sc(T, H, NP):
    info = plsc.get_sparse_core_info()
    nw = info.num_cores * info.num_subcores
    rows = T // nw
    ch = 32
    nch = rows // ch
    nsl = H // 16
    mesh = plsc.VectorSubcoreMesh(core_axis_name="c", subcore_axis_name="s")

    @functools.partial(
        pl.kernel, mesh=mesh,
        out_type=jax.ShapeDtypeStruct((T, H), jnp.float32),
        scratch_types=[
            pltpu.VMEM((ch,), jnp.int32),
            pltpu.VMEM((ch, H), jnp.float32),
            pltpu.VMEM((ch, H), jnp.float32),
            pltpu.SemaphoreType.DMA,
        ],
    )
    def k(y_hbm, pos0_hbm, pos1_hbm, out_hbm, idx_v, y_v, o_v, sem):
        wid = lax.axis_index("s") * info.num_cores + lax.axis_index("c")
        for c in range(nch):
            bc = wid * rows + c * ch
            pltpu.sync_copy(pos0_hbm.at[pl.ds(bc, ch)], idx_v)
            pltpu.async_copy(y_hbm.at[idx_v], o_v, sem).wait()
            pltpu.sync_copy(pos1_hbm.at[pl.ds(bc, ch)], idx_v)
            pltpu.async_copy(y_hbm.at[idx_v], y_v, sem).wait()

            def row(r, _):
                def col(kk, _):
                    for u in range(4):
                        sl = pl.ds((kk * 4 + u) * 16, 16)
                        o_v[r, sl] = o_v[r, sl] + y_v[r, sl]
                    return 0
                return lax.fori_loop(0, nsl // 4, col, 0)

            lax.fori_loop(0, ch, row, 0)
            pltpu.sync_copy(o_v, out_hbm.at[pl.ds(bc, ch)])

    return k


def kernel(hidden_states, router_logits, w1, b1, w2, b2):
    T, H = hidden_states.shape
    E = router_logits.shape[1]
    F = w1.shape[2]
    nt = (T * 2 + E * (_BT - 1)) // _BT
    NP = nt * _BT
    nf = F // _BF

    pos0, pos1, g0, g1, te, grp = _make_routing(T, E, nt)(router_logits)
    pos0 = pos0.reshape(T)
    pos1 = pos1.reshape(T)
    te = te.reshape(nt)
    grp = grp.reshape(nt)

    xs, gs = _dispatch_sc(T, H, NP)(hidden_states, pos0, pos1, g0, g1)

    b1r = b1.reshape(E, 1, F)
    b2r = b2.reshape(E, 1, H)
    y = _make_gemm(NP, H, F, E, nt, nf)(te, grp, xs, b1r, b2r, gs, w1, w2)

    out = _combine_sc(T, H, NP)(y, pos0, pos1)
    return out


# speedup vs baseline: 1.3545x; 1.0084x over previous
import functools

import jax
import jax.numpy as jnp
from jax import lax
from jax.experimental import pallas as pl
from jax.experimental.pallas import tpu as pltpu
from jax.experimental.pallas import tpu_sc as plsc

_BT = 256
_BF = 512
_CB = 256


def _routing_body(logits_ref, pos0_ref, pos1_ref, g0_ref, g1_ref, te_ref,
                  grp_ref, *, bt, nt):
    l = logits_ref[...]
    T, E = l.shape
    ie = lax.broadcasted_iota(jnp.int32, (T, E), 1)
    m1 = jnp.max(l, axis=1, keepdims=True)
    idx1 = jnp.min(jnp.where(l == m1, ie, E), axis=1, keepdims=True)
    sel1 = ie == idx1
    l2 = jnp.where(sel1, -jnp.inf, l)
    m2 = jnp.max(l2, axis=1, keepdims=True)
    idx2 = jnp.min(jnp.where(l2 == m2, ie, E), axis=1, keepdims=True)
    sel2 = ie == idx2
    ed = jnp.exp(m2 - m1)
    g1 = 1.0 / (1.0 + ed)
    g2 = ed * g1

    oh1 = jnp.where(sel1, 1.0, 0.0)
    oh2 = jnp.where(sel2, 1.0, 0.0)
    oh = oh1 + oh2

    nb = T // _CB
    ir = lax.broadcasted_iota(jnp.int32, (_CB, _CB), 0)
    ic = lax.broadcasted_iota(jnp.int32, (_CB, _CB), 1)
    tri = jnp.where(ir > ic, 1.0, 0.0)
    blocks = []
    carry = jnp.zeros((1, E), jnp.float32)
    for b in range(nb):
        blk = oh[b * _CB:(b + 1) * _CB]
        blocks.append(jnp.dot(tri, blk, preferred_element_type=jnp.float32)
                      + carry)
        carry = carry + jnp.sum(blk, axis=0, keepdims=True)
    S = jnp.concatenate(blocks, axis=0)
    counts = carry

    fbt = jnp.float32(bt)
    cpad = jnp.floor((counts + (fbt - 1.0)) / fbt) * fbt
    iu = lax.broadcasted_iota(jnp.int32, (E, E), 0)
    ju = lax.broadcasted_iota(jnp.int32, (E, E), 1)
    ut = jnp.where(iu < ju, 1.0, 0.0)
    off = jnp.dot(cpad, ut, preferred_element_type=jnp.float32)
    off_end = off + cpad

    offb = jnp.broadcast_to(off, (T, E))
    pos0 = jnp.sum(jnp.where(sel1, offb + S, 0.0), axis=1, keepdims=True)
    pos1 = jnp.sum(jnp.where(sel2, offb + S, 0.0), axis=1, keepdims=True)
    pos0_ref[...] = pos0.astype(jnp.int32)
    pos1_ref[...] = pos1.astype(jnp.int32)

    g0_ref[...] = jnp.broadcast_to(g1, (T, 128))
    g1_ref[...] = jnp.broadcast_to(g2, (T, 128))

    tb = lax.broadcasted_iota(jnp.int32, (nt, E), 0).astype(jnp.float32) * fbt
    oeb = jnp.broadcast_to(off_end, (nt, E))
    te = jnp.sum(jnp.where(oeb <= tb, 1, 0), axis=1, keepdims=True)
    te = jnp.minimum(te, E - 1)
    te_ref[...] = te.astype(jnp.int32)

    tprev = jnp.concatenate([te[:1], te[:-1]], axis=0)
    chg = jnp.where(te != tprev, 1.0, 0.0)
    gr = lax.broadcasted_iota(jnp.int32, (nt, nt), 0)
    gc = lax.broadcasted_iota(jnp.int32, (nt, nt), 1)
    tri_i = jnp.where(gr >= gc, 1.0, 0.0)
    grp = jnp.dot(tri_i, chg, preferred_element_type=jnp.float32)
    grp_ref[...] = grp.astype(jnp.int32)


def _make_routing(T, E, nt):
    return pl.pallas_call(
        functools.partial(_routing_body, bt=_BT, nt=nt),
        out_shape=[
            jax.ShapeDtypeStruct((T, 1), jnp.int32),
            jax.ShapeDtypeStruct((T, 1), jnp.int32),
            jax.ShapeDtypeStruct((T, 128), jnp.float32),
            jax.ShapeDtypeStruct((T, 128), jnp.float32),
            jax.ShapeDtypeStruct((nt, 1), jnp.int32),
            jax.ShapeDtypeStruct((nt, 1), jnp.int32),
        ],
    )


def _dispatch_sc(T, H, NP):
    info = plsc.get_sparse_core_info()
    nw = info.num_cores * info.num_subcores
    rows = T // nw
    mesh = plsc.VectorSubcoreMesh(core_axis_name="c", subcore_axis_name="s")

    @functools.partial(
        pl.kernel, mesh=mesh,
        out_type=[jax.ShapeDtypeStruct((NP, H), jnp.float32),
                  jax.ShapeDtypeStruct((NP, 128), jnp.float32)],
        scratch_types=[
            pltpu.VMEM((rows,), jnp.int32),
            pltpu.VMEM((rows,), jnp.int32),
            pltpu.VMEM((rows, H), jnp.float32),
            pltpu.VMEM((rows, 128), jnp.float32),
            pltpu.VMEM((rows, 128), jnp.float32),
            pltpu.SemaphoreType.DMA,
        ],
    )
    def k(x_hbm, pos0_hbm, pos1_hbm, g0_hbm, g1_hbm, xs_hbm, gs_hbm,
          idx0_v, idx1_v, x_v, g0_v, g1_v, sem):
        wid = lax.axis_index("s") * info.num_cores + lax.axis_index("c")
        base = wid * rows
        pltpu.sync_copy(pos0_hbm.at[pl.ds(base, rows)], idx0_v)
        pltpu.sync_copy(pos1_hbm.at[pl.ds(base, rows)], idx1_v)
        pltpu.sync_copy(x_hbm.at[pl.ds(base, rows)], x_v)
        pltpu.sync_copy(g0_hbm.at[pl.ds(base, rows)], g0_v)
        pltpu.sync_copy(g1_hbm.at[pl.ds(base, rows)], g1_v)
        c1 = pltpu.async_copy(x_v, xs_hbm.at[idx0_v], sem)
        c2 = pltpu.async_copy(x_v, xs_hbm.at[idx1_v], sem)
        c3 = pltpu.async_copy(g0_v, gs_hbm.at[idx0_v], sem)
        c4 = pltpu.async_copy(g1_v, gs_hbm.at[idx1_v], sem)
        c1.wait()
        c2.wait()
        c3.wait()
        c4.wait()

    return k


_NBUF = 4
_LOOK = 3


def _gemm_body(te_ref, grp_ref, x_ref, b1_ref, b2_ref, gs_ref,
               w1_ref, w2_ref, out_ref, acc_ref, xb_ref, wb1_ref, wb2_ref,
               sems, *, nf, nt, bt, bf):
    f = pl.program_id(0)
    i = pl.program_id(1)
    isl = pl.ds(i * bt, bt)
    G = grp_ref[nt - 1] + 1
    s = f * nt + i
    S = nf * nt

    def dma_pair(f2, i2, slot):
        e2 = te_ref[i2]
        c1 = pltpu.make_async_copy(
            w1_ref.at[e2, :, pl.ds(f2 * bf, bf)], wb1_ref.at[slot],
            sems.at[slot])
        c2 = pltpu.make_async_copy(
            w2_ref.at[e2, pl.ds(f2 * bf, bf), :], wb2_ref.at[slot],
            sems.at[slot])
        return c1, c2

    def issue(f2, i2, dk2):
        slot = lax.rem(dk2, _NBUF)
        c1, c2 = dma_pair(f2, i2, slot)
        c1.start()
        c2.start()

    @pl.when(s == 0)
    def _():
        issue(0, 0, grp_ref[0])
        for u in range(1, _LOOK):
            @pl.when(grp_ref[u] != grp_ref[u - 1])
            def _(u=u):
                issue(0, u, grp_ref[u])

    s2 = s + _LOOK

    @pl.when(s2 < S)
    def _():
        f2 = s2 // nt
        i2 = lax.rem(s2, nt)
        dk2 = f2 * G + grp_ref[i2]
        s3 = s2 - 1
        f3 = s3 // nt
        i3 = lax.rem(s3, nt)
        dk3 = f3 * G + grp_ref[i3]

        @pl.when(dk2 != dk3)
        def _():
            issue(f2, i2, dk2)

    dk = f * G + grp_ref[i]
    slot = lax.rem(dk, _NBUF)
    sp = jnp.maximum(s - 1, 0)
    fp = sp // nt
    ip = lax.rem(sp, nt)
    dkp = jnp.where(s == 0, -1, fp * G + grp_ref[ip])

    @pl.when(dk != dkp)
    def _():
        c1, c2 = dma_pair(f, i, slot)
        c1.wait()
        c2.wait()

    @pl.when(f == 0)
    def _():
        xb_ref[isl, :] = x_ref[...].astype(jnp.bfloat16)

    x = xb_ref[isl, :]
    h = jnp.dot(x, wb1_ref[slot], preferred_element_type=jnp.float32) \
        + b1_ref[0]
    h = jax.nn.gelu(h)
    part = jnp.dot(h, wb2_ref[slot], preferred_element_type=jnp.float32)

    total = part + jnp.where(f == 0, jnp.zeros_like(part), acc_ref[isl, :])
    acc_ref[isl, :] = total

    @pl.when(f == nf - 1)
    def _():
        c = gs_ref[:, :1]
        out_ref[...] = c * (total + b2_ref[0])


def _make_gemm(NP, H, F, E, nt, nf):
    grid_spec = pltpu.PrefetchScalarGridSpec(
        num_scalar_prefetch=2,
        grid=(nf, nt),
        in_specs=[
            pl.BlockSpec(
                (_BT, H),
                lambda f, i, te, g: (jnp.where(f == 0, i, 0), 0)),
            pl.BlockSpec((1, 1, _BF), lambda f, i, te, g: (te[i], 0, f)),
            pl.BlockSpec((1, 1, H), lambda f, i, te, g: (te[i], 0, 0)),
            pl.BlockSpec(
                (_BT, 128),
                lambda f, i, te, g: (jnp.where(f == nf - 1, i, 0), 0)),
            pl.BlockSpec(memory_space=pl.ANY),
            pl.BlockSpec(memory_space=pl.ANY),
        ],
        out_specs=pl.BlockSpec(
            (_BT, H), lambda f, i, te, g: (jnp.where(f == nf - 1, i, 0), 0)),
        scratch_shapes=[
            pltpu.VMEM((NP, H), jnp.float32),
            pltpu.VMEM((NP, H), jnp.bfloat16),
            pltpu.VMEM((_NBUF, H, _BF), jnp.float32),
            pltpu.VMEM((_NBUF, _BF, H), jnp.float32),
            pltpu.SemaphoreType.DMA((_NBUF,)),
        ],
    )
    return pl.pallas_call(
        functools.partial(_gemm_body, nf=nf, nt=nt, bt=_BT, bf=_BF),
        grid_spec=grid_spec,
        out_shape=jax.ShapeDtypeStruct((NP, H), jnp.float32),
        compiler_params=pltpu.CompilerParams(
            dimension_semantics=("arbitrary", "arbitrary"),
        ),
    )


def _combine_sc(T, H, NP):
    info = plsc.get_sparse_core_info()
    nw = info.num_cores * info.num_subcores
    rows = T // nw
    ch = 32
    nch = rows // ch
    nsl = H // 16
    mesh = plsc.VectorSubcoreMesh(core_axis_name="c", subcore_axis_name="s")

    @functools.partial(
        pl.kernel, mesh=mesh,
        out_type=jax.ShapeDtypeStruct((T, H), jnp.float32),
        scratch_types=[
            pltpu.VMEM((ch,), jnp.int32),
            pltpu.VMEM((ch, H), jnp.float32),
            pltpu.VMEM((ch, H), jnp.float32),
            pltpu.SemaphoreType.DMA,
        ],
    )
    def k(y_hbm, pos0_hbm, pos1_hbm, out_hbm, idx_v, y_v, o_v, sem):
        wid = lax.axis_index("s") * info.num_cores + lax.axis_index("c")
        for c in range(nch):
            bc = wid * rows + c * ch
            pltpu.sync_copy(pos0_hbm.at[pl.ds(bc, ch)], idx_v)
            pltpu.async_copy(y_hbm.at[idx_v], o_v, sem).wait()
            pltpu.sync_copy(pos1_hbm.at[pl.ds(bc, ch)], idx_v)
            pltpu.async_copy(y_hbm.at[idx_v], y_v, sem).wait()

            def row(r, _):
                def col(kk, _):
                    for u in range(4):
                        sl = pl.ds((kk * 4 + u) * 16, 16)
                        o_v[r, sl] = o_v[r, sl] + y_v[r, sl]
                    return 0
                return lax.fori_loop(0, nsl // 4, col, 0)

            lax.fori_loop(0, ch, row, 0)
            pltpu.sync_copy(o_v, out_hbm.at[pl.ds(bc, ch)])

    return k


def kernel(hidden_states, router_logits, w1, b1, w2, b2):
    T, H = hidden_states.shape
    E = router_logits.shape[1]
    F = w1.shape[2]
    nt = (T * 2 + E * (_BT - 1)) // _BT
    NP = nt * _BT
    nf = F // _BF

    pos0, pos1, g0, g1, te, grp = _make_routing(T, E, nt)(router_logits)
    pos0 = pos0.reshape(T)
    pos1 = pos1.reshape(T)
    te = te.reshape(nt)
    grp = grp.reshape(nt)

    xs, gs = _dispatch_sc(T, H, NP)(hidden_states, pos0, pos1, g0, g1)

    b1r = b1.reshape(E, 1, F)
    b2r = b2.reshape(E, 1, H)
    y = _make_gemm(NP, H, F, E, nt, nf)(te, grp, xs, b1r, b2r, gs, w1, w2)

    out = _combine_sc(T, H, NP)(y, pos0, pos1)
    return out
